# Initial kernel scaffold; baseline (speedup 1.0000x reference)
import jax
import jax.numpy as jnp
from jax.experimental import pallas as pl


def kernel(x, edge_index, edge_weight, W_in, b_in, W_gcn, b_gcn, W_out, b_out):
    raise NotImplementedError("write your pallas kernel here")


# speedup vs baseline: 13.7509x; 13.7509x over previous
import functools

import jax
import jax.numpy as jnp
from jax import lax
from jax.experimental import pallas as pl
from jax.experimental.pallas import tpu as pltpu
from jax.experimental.pallas import tpu_sc as plsc

N = 10000
D = 128
H = 128

NC = 2
NS = 16
L = 16
NW = NC * NS

K = 128
N_PAD = NS * 640
ROWS_PER_TILE = N_PAD // NS

E_CH = 80
BLKB = 8
NBLKB = E_CH // BLKB
BLKA = 16
NBLKA = E_CH // BLKA
E_PAD = NW * E_CH * K
E2D_ROWS = E_PAD // K


def _dense_in_body(x_ref, w_in_ref, b_in_ref, w_gcn_ref, h2_ref):
    h = jnp.dot(x_ref[...], w_in_ref[...], preferred_element_type=jnp.float32)
    h = jnp.maximum(h + b_in_ref[...], 0.0)
    h2_ref[...] = jnp.dot(h, w_gcn_ref[...], preferred_element_type=jnp.float32)


def _norm_body(degp_ref, h2_ref, dinv_ref, g_ref):
    deg = degp_ref[0] + degp_ref[1] + 1.0
    dv = lax.rsqrt(deg)
    dinv_ref[...] = dv
    g_ref[...] = h2_ref[...] * dv


def _combine_body(acc_ref, g_ref, dinv_ref, bgcn_ref, wout_ref, bout_ref,
                  out_ref):
    dv = dinv_ref[...]
    t = dv * (acc_ref[0] + acc_ref[1] + g_ref[...]) + bgcn_ref[...]
    t = jnp.maximum(t, 0.0)
    out_ref[...] = (
        jnp.dot(t, wout_ref[...], preferred_element_type=jnp.float32)
        + bout_ref[...])


def _sc_deg_body(col_hbm, ew_hbm,
                 deg_out,
                 deg_sh,
                 zrow, colblk, ewblk,
                 isem, dsem):
    cidx = lax.axis_index("c")
    sidx = lax.axis_index("s")
    tile_row0 = (sidx * NC + cidx) * E_CH

    def load_block(b, bb):
        base = tile_row0 + b * BLKA
        pltpu.async_copy(col_hbm.at[pl.ds(base, BLKA)], colblk.at[bb], isem)
        pltpu.async_copy(ew_hbm.at[pl.ds(base, BLKA)], ewblk.at[bb], isem)

    def wait_block():
        pltpu.make_async_copy(col_hbm.at[pl.ds(0, BLKA)], colblk.at[0],
                              isem).wait()
        pltpu.make_async_copy(ew_hbm.at[pl.ds(0, BLKA)], ewblk.at[0],
                              isem).wait()

    load_block(0, 0)
    zvec = jnp.zeros((L,), jnp.float32)
    for f in range(K // L):
        zrow[pl.ds(f * L, L)] = zvec
    for p in range(ROWS_PER_TILE // K):
        pltpu.sync_copy(zrow,
                        deg_sh.at[pl.ds(sidx * ROWS_PER_TILE + p * K, K)])
    plsc.subcore_barrier()

    def _deg_block(b, _):
        bb = lax.rem(b, 2)
        wait_block()

        @pl.when(b + 1 < NBLKA)
        def _():
            load_block(b + 1, 1 - bb)

        def _fire(j, _):
            pltpu.async_copy(ewblk.at[bb, j], deg_sh.at[colblk.at[bb, j]],
                             dsem, add=True)
            return 0

        lax.fori_loop(0, BLKA, _fire, 0)

        def _drain(j, _):
            pltpu.make_async_copy(ewblk.at[0, 0], deg_sh.at[colblk.at[0, 0]],
                                  dsem).wait()
            return 0

        lax.fori_loop(0, BLKA, _drain, 0)
        return 0

    lax.fori_loop(0, NBLKA, _deg_block, 0)
    plsc.subcore_barrier()
    pltpu.sync_copy(deg_sh.at[pl.ds(sidx * ROWS_PER_TILE, ROWS_PER_TILE)],
                    deg_out.at[cidx, pl.ds(sidx * ROWS_PER_TILE,
                                           ROWS_PER_TILE)])


def _sc_edge_body(row_hbm, col_hbm, ew_hbm, g_hbm,
                  acc_out,
                  acc_sh,
                  msg_v, rowblk, colblk, ewblk,
                  isem, g0, g1, s0, s1):
    cidx = lax.axis_index("c")
    sidx = lax.axis_index("s")
    wid = sidx * NC + cidx
    gsem = (g0, g1)
    ssem = (s0, s1)
    tile_row0 = wid * E_CH

    def load_block(b, bb):
        base = tile_row0 + b * BLKB
        pltpu.async_copy(row_hbm.at[pl.ds(base, BLKB)], rowblk.at[bb], isem)
        pltpu.async_copy(col_hbm.at[pl.ds(base, BLKB)], colblk.at[bb], isem)
        pltpu.async_copy(ew_hbm.at[pl.ds(base, BLKB)], ewblk.at[bb], isem)

    def wait_block():
        pltpu.make_async_copy(row_hbm.at[pl.ds(0, BLKB)], rowblk.at[0],
                              isem).wait()
        pltpu.make_async_copy(col_hbm.at[pl.ds(0, BLKB)], colblk.at[0],
                              isem).wait()
        pltpu.make_async_copy(ew_hbm.at[pl.ds(0, BLKB)], ewblk.at[0],
                              isem).wait()

    load_block(0, 0)
    zvec = jnp.zeros((L,), jnp.float32)

    def _zrow(e, _):
        for f in range(H // L):
            msg_v[0, e, pl.ds(f * L, L)] = zvec
        return 0

    lax.fori_loop(0, K, _zrow, 0)
    for p in range(ROWS_PER_TILE // K):
        pltpu.sync_copy(msg_v.at[0],
                        acc_sh.at[pl.ds(sidx * ROWS_PER_TILE + p * K, K)])
    plsc.subcore_barrier()

    wait_block()
    pltpu.async_copy(g_hbm.at[rowblk.at[0, 0]], msg_v.at[0], gsem[0])

    def _edge_block(b, _):
        bb = lax.rem(b, 2)

        for j in range(BLKB):
            p = j % 2
            q = 1 - p
            pltpu.make_async_copy(g_hbm.at[rowblk.at[bb, j]], msg_v.at[p],
                                  gsem[p]).wait()

            if j == 0:
                @pl.when(b > 0)
                def _():
                    pltpu.make_async_copy(
                        msg_v.at[q], acc_sh.at[colblk.at[1 - bb, BLKB - 1]],
                        ssem[q]).wait()

                @pl.when(b + 1 < NBLKB)
                def _():
                    load_block(b + 1, 1 - bb)
            else:
                pltpu.make_async_copy(msg_v.at[q],
                                      acc_sh.at[colblk.at[bb, j - 1]],
                                      ssem[q]).wait()

            if j == BLKB // 2:
                @pl.when(b + 1 < NBLKB)
                def _():
                    wait_block()

            if j + 1 < BLKB:
                pltpu.async_copy(g_hbm.at[rowblk.at[bb, j + 1]], msg_v.at[q],
                                 gsem[q])
            else:
                @pl.when(b + 1 < NBLKB)
                def _():
                    pltpu.async_copy(g_hbm.at[rowblk.at[1 - bb, 0]],
                                     msg_v.at[q], gsem[q])

            def _scale_group(g, _):
                svec = ewblk[bb, j, pl.ds(g * L, L)]
                for i in range(L):
                    e = g * L + i
                    s = svec[i]
                    for f in range(H // L):
                        msg_v[p, e, pl.ds(f * L, L)] = (
                            msg_v[p, e, pl.ds(f * L, L)] * s)
                return 0

            lax.fori_loop(0, K // L, _scale_group, 0)

            pltpu.async_copy(msg_v.at[p], acc_sh.at[colblk.at[bb, j]],
                             ssem[p], add=True)
        return 0

    lax.fori_loop(0, NBLKB, _edge_block, 0)
    pltpu.make_async_copy(msg_v.at[(BLKB - 1) % 2], acc_sh.at[colblk.at[0, 0]],
                          ssem[(BLKB - 1) % 2]).wait()
    plsc.subcore_barrier()

    pltpu.sync_copy(acc_sh.at[pl.ds(sidx * ROWS_PER_TILE, ROWS_PER_TILE)],
                    acc_out.at[cidx, pl.ds(sidx * ROWS_PER_TILE,
                                           ROWS_PER_TILE)])


_MESH = plsc.VectorSubcoreMesh(core_axis_name="c", subcore_axis_name="s")
_SC_PARAMS = pltpu.CompilerParams(needs_layout_passes=False)

_sc_deg = functools.partial(
    pl.kernel,
    out_type=jax.ShapeDtypeStruct((NC, N_PAD), jnp.float32),
    mesh=_MESH,
    compiler_params=_SC_PARAMS,
    scratch_types=[
        pltpu.VMEM_SHARED((N_PAD,), jnp.float32),
        pltpu.VMEM((K,), jnp.float32),
        pltpu.VMEM((2, BLKA, K), jnp.int32),
        pltpu.VMEM((2, BLKA, K), jnp.float32),
        pltpu.SemaphoreType.DMA,
        pltpu.SemaphoreType.DMA,
    ],
)(_sc_deg_body)

_sc_edge = functools.partial(
    pl.kernel,
    out_type=jax.ShapeDtypeStruct((NC, N_PAD, H), jnp.float32),
    mesh=_MESH,
    compiler_params=_SC_PARAMS,
    scratch_types=[
        pltpu.VMEM_SHARED((N_PAD, H), jnp.float32),
        pltpu.VMEM((2, K, H), jnp.float32),
        pltpu.VMEM((2, BLKB, K), jnp.int32),
        pltpu.VMEM((2, BLKB, K), jnp.int32),
        pltpu.VMEM((2, BLKB, K), jnp.float32),
        pltpu.SemaphoreType.DMA,
        pltpu.SemaphoreType.DMA,
        pltpu.SemaphoreType.DMA,
        pltpu.SemaphoreType.DMA,
        pltpu.SemaphoreType.DMA,
    ],
)(_sc_edge_body)


def kernel(x, edge_index, edge_weight, W_in, b_in, W_gcn, b_gcn, W_out, b_out):
    x_p = jnp.pad(x, ((0, N_PAD - N), (0, 0)))
    pad_e = E_PAD - edge_index.shape[1]
    row_p = jnp.concatenate(
        [edge_index[0], jnp.zeros((pad_e,), jnp.int32)]).reshape(E2D_ROWS, K)
    col_p = jnp.concatenate(
        [edge_index[1], jnp.zeros((pad_e,), jnp.int32)]).reshape(E2D_ROWS, K)
    ew_p = jnp.concatenate(
        [edge_weight, jnp.zeros((pad_e,), jnp.float32)]).reshape(E2D_ROWS, K)

    BR = 1024

    h2 = pl.pallas_call(
        _dense_in_body,
        grid=(N_PAD // BR,),
        in_specs=[
            pl.BlockSpec((BR, D), lambda i: (i, 0)),
            pl.BlockSpec((D, H), lambda i: (0, 0)),
            pl.BlockSpec((1, H), lambda i: (0, 0)),
            pl.BlockSpec((H, H), lambda i: (0, 0)),
        ],
        out_specs=pl.BlockSpec((BR, H), lambda i: (i, 0)),
        out_shape=jax.ShapeDtypeStruct((N_PAD, H), jnp.float32),
    )(x_p, W_in, b_in.reshape(1, H), W_gcn)

    degp = _sc_deg(col_p, ew_p)

    dinv, g = pl.pallas_call(
        _norm_body,
        grid=(N_PAD // BR,),
        in_specs=[
            pl.BlockSpec((NC, BR, 1), lambda i: (0, i, 0)),
            pl.BlockSpec((BR, H), lambda i: (i, 0)),
        ],
        out_specs=[
            pl.BlockSpec((BR, 1), lambda i: (i, 0)),
            pl.BlockSpec((BR, H), lambda i: (i, 0)),
        ],
        out_shape=[
            jax.ShapeDtypeStruct((N_PAD, 1), jnp.float32),
            jax.ShapeDtypeStruct((N_PAD, H), jnp.float32),
        ],
    )(degp.reshape(NC, N_PAD, 1), h2)

    acc = _sc_edge(row_p, col_p, ew_p, g)

    out = pl.pallas_call(
        _combine_body,
        grid=(N_PAD // BR,),
        in_specs=[
            pl.BlockSpec((NC, BR, H), lambda i: (0, i, 0)),
            pl.BlockSpec((BR, H), lambda i: (i, 0)),
            pl.BlockSpec((BR, 1), lambda i: (i, 0)),
            pl.BlockSpec((1, H), lambda i: (0, 0)),
            pl.BlockSpec((H, 40), lambda i: (0, 0)),
            pl.BlockSpec((1, 40), lambda i: (0, 0)),
        ],
        out_specs=pl.BlockSpec((BR, 40), lambda i: (i, 0)),
        out_shape=jax.ShapeDtypeStruct((N_PAD, 40), jnp.float32),
    )(acc, g, dinv, b_gcn.reshape(1, H), W_out, b_out.reshape(1, 40))

    return out[:N]


# speedup vs baseline: 15.5203x; 1.1287x over previous
import functools

import jax
import jax.numpy as jnp
from jax import lax
from jax.experimental import pallas as pl
from jax.experimental.pallas import tpu as pltpu
from jax.experimental.pallas import tpu_sc as plsc

N = 10000
D = 128
H = 128

NC = 2
NS = 16
L = 16
NW = NC * NS

K = 128
N_PAD = NS * 640
ROWS_PER_TILE = N_PAD // NS

E_CH0 = 128
E_CH1 = 32
BLKB = 8
NBLKB0 = E_CH0 // BLKB
NBLKB1 = E_CH1 // BLKB
BLKA = 16
NBLKA0 = E_CH0 // BLKA
NBLKA1 = E_CH1 // BLKA
E_PAD = NS * (E_CH0 + E_CH1) * K
E2D_ROWS = E_PAD // K


def _dense_in_body(x_ref, w_in_ref, b_in_ref, w_gcn_ref, h2_ref):
    h = jnp.dot(x_ref[...], w_in_ref[...], preferred_element_type=jnp.float32)
    h = jnp.maximum(h + b_in_ref[...], 0.0)
    h2_ref[...] = jnp.dot(h, w_gcn_ref[...], preferred_element_type=jnp.float32)


def _norm_body(degp_ref, h2_ref, dinv_ref, g_ref):
    deg = degp_ref[0] + degp_ref[1] + 1.0
    dv = lax.rsqrt(deg)
    dinv_ref[...] = dv
    g_ref[...] = h2_ref[...] * dv


def _combine_body(acc_ref, g_ref, dinv_ref, bgcn_ref, wout_ref, bout_ref,
                  out_ref):
    dv = dinv_ref[...]
    t = dv * (acc_ref[0] + acc_ref[1] + g_ref[...]) + bgcn_ref[...]
    t = jnp.maximum(t, 0.0)
    out_ref[...] = (
        jnp.dot(t, wout_ref[...], preferred_element_type=jnp.float32)
        + bout_ref[...])


def _sc_deg_body(col_hbm, ew_hbm,
                 deg_out,
                 deg_sh,
                 zrow, colblk, ewblk,
                 isem, dsem):
    cidx = lax.axis_index("c")
    sidx = lax.axis_index("s")
    tile_row0 = jnp.where(cidx == 0, sidx * E_CH0, NS * E_CH0 + sidx * E_CH1)
    nblk = jnp.where(cidx == 0, NBLKA0, NBLKA1)

    def load_block(b, bb):
        base = tile_row0 + b * BLKA
        pltpu.async_copy(col_hbm.at[pl.ds(base, BLKA)], colblk.at[bb], isem)
        pltpu.async_copy(ew_hbm.at[pl.ds(base, BLKA)], ewblk.at[bb], isem)

    def wait_block():
        pltpu.make_async_copy(col_hbm.at[pl.ds(0, BLKA)], colblk.at[0],
                              isem).wait()
        pltpu.make_async_copy(ew_hbm.at[pl.ds(0, BLKA)], ewblk.at[0],
                              isem).wait()

    load_block(0, 0)
    zvec = jnp.zeros((L,), jnp.float32)
    for f in range(K // L):
        zrow[pl.ds(f * L, L)] = zvec
    for p in range(ROWS_PER_TILE // K):
        pltpu.sync_copy(zrow,
                        deg_sh.at[pl.ds(sidx * ROWS_PER_TILE + p * K, K)])
    plsc.subcore_barrier()

    def _deg_block(b, _):
        bb = lax.rem(b, 2)
        wait_block()

        @pl.when(b + 1 < nblk)
        def _():
            load_block(b + 1, 1 - bb)

        def _fire(j, _):
            pltpu.async_copy(ewblk.at[bb, j], deg_sh.at[colblk.at[bb, j]],
                             dsem, add=True)
            return 0

        lax.fori_loop(0, BLKA, _fire, 0)

        def _drain(j, _):
            pltpu.make_async_copy(ewblk.at[0, 0], deg_sh.at[colblk.at[0, 0]],
                                  dsem).wait()
            return 0

        lax.fori_loop(0, BLKA, _drain, 0)
        return 0

    lax.fori_loop(0, nblk, _deg_block, 0)
    plsc.subcore_barrier()
    pltpu.sync_copy(deg_sh.at[pl.ds(sidx * ROWS_PER_TILE, ROWS_PER_TILE)],
                    deg_out.at[cidx, pl.ds(sidx * ROWS_PER_TILE,
                                           ROWS_PER_TILE)])


def _sc_edge_body(row_hbm, col_hbm, ew_hbm, g_hbm,
                  acc_out,
                  acc_sh,
                  msg_v, rowblk, colblk, ewblk,
                  isem, g0, g1, s0, s1):
    cidx = lax.axis_index("c")
    sidx = lax.axis_index("s")
    gsem = (g0, g1)
    ssem = (s0, s1)
    tile_row0 = jnp.where(cidx == 0, sidx * E_CH0, NS * E_CH0 + sidx * E_CH1)
    nblk = jnp.where(cidx == 0, NBLKB0, NBLKB1)

    def load_block(b, bb):
        base = tile_row0 + b * BLKB
        pltpu.async_copy(row_hbm.at[pl.ds(base, BLKB)], rowblk.at[bb], isem)
        pltpu.async_copy(col_hbm.at[pl.ds(base, BLKB)], colblk.at[bb], isem)
        pltpu.async_copy(ew_hbm.at[pl.ds(base, BLKB)], ewblk.at[bb], isem)

    def wait_block():
        pltpu.make_async_copy(row_hbm.at[pl.ds(0, BLKB)], rowblk.at[0],
                              isem).wait()
        pltpu.make_async_copy(col_hbm.at[pl.ds(0, BLKB)], colblk.at[0],
                              isem).wait()
        pltpu.make_async_copy(ew_hbm.at[pl.ds(0, BLKB)], ewblk.at[0],
                              isem).wait()

    load_block(0, 0)
    zvec = jnp.zeros((L,), jnp.float32)

    def _zrow(e, _):
        for f in range(H // L):
            msg_v[0, e, pl.ds(f * L, L)] = zvec
        return 0

    lax.fori_loop(0, K, _zrow, 0)
    for p in range(ROWS_PER_TILE // K):
        pltpu.sync_copy(msg_v.at[0],
                        acc_sh.at[pl.ds(sidx * ROWS_PER_TILE + p * K, K)])
    plsc.subcore_barrier()

    wait_block()
    pltpu.async_copy(g_hbm.at[rowblk.at[0, 0]], msg_v.at[0], gsem[0])

    def _edge_block(b, _):
        bb = lax.rem(b, 2)

        for j in range(BLKB):
            p = j % 2
            q = 1 - p
            pltpu.make_async_copy(g_hbm.at[rowblk.at[bb, j]], msg_v.at[p],
                                  gsem[p]).wait()

            if j == 0:
                @pl.when(b > 0)
                def _():
                    pltpu.make_async_copy(
                        msg_v.at[q], acc_sh.at[colblk.at[1 - bb, BLKB - 1]],
                        ssem[q]).wait()

                @pl.when(b + 1 < nblk)
                def _():
                    load_block(b + 1, 1 - bb)
            else:
                pltpu.make_async_copy(msg_v.at[q],
                                      acc_sh.at[colblk.at[bb, j - 1]],
                                      ssem[q]).wait()

            if j == BLKB // 2:
                @pl.when(b + 1 < nblk)
                def _():
                    wait_block()

            if j + 1 < BLKB:
                pltpu.async_copy(g_hbm.at[rowblk.at[bb, j + 1]], msg_v.at[q],
                                 gsem[q])
            else:
                @pl.when(b + 1 < nblk)
                def _():
                    pltpu.async_copy(g_hbm.at[rowblk.at[1 - bb, 0]],
                                     msg_v.at[q], gsem[q])

            def _scale_group(g, _):
                svec = ewblk[bb, j, pl.ds(g * L, L)]
                for i in range(L):
                    e = g * L + i
                    s = svec[i]
                    for f in range(H // L):
                        msg_v[p, e, pl.ds(f * L, L)] = (
                            msg_v[p, e, pl.ds(f * L, L)] * s)
                return 0

            lax.fori_loop(0, K // L, _scale_group, 0)

            pltpu.async_copy(msg_v.at[p], acc_sh.at[colblk.at[bb, j]],
                             ssem[p], add=True)
        return 0

    lax.fori_loop(0, nblk, _edge_block, 0)
    pltpu.make_async_copy(msg_v.at[(BLKB - 1) % 2], acc_sh.at[colblk.at[0, 0]],
                          ssem[(BLKB - 1) % 2]).wait()
    plsc.subcore_barrier()

    pltpu.sync_copy(acc_sh.at[pl.ds(sidx * ROWS_PER_TILE, ROWS_PER_TILE)],
                    acc_out.at[cidx, pl.ds(sidx * ROWS_PER_TILE,
                                           ROWS_PER_TILE)])


_MESH = plsc.VectorSubcoreMesh(core_axis_name="c", subcore_axis_name="s")
_SC_PARAMS = pltpu.CompilerParams(needs_layout_passes=False)

_sc_deg = functools.partial(
    pl.kernel,
    out_type=jax.ShapeDtypeStruct((NC, N_PAD), jnp.float32),
    mesh=_MESH,
    compiler_params=_SC_PARAMS,
    scratch_types=[
        pltpu.VMEM_SHARED((N_PAD,), jnp.float32),
        pltpu.VMEM((K,), jnp.float32),
        pltpu.VMEM((2, BLKA, K), jnp.int32),
        pltpu.VMEM((2, BLKA, K), jnp.float32),
        pltpu.SemaphoreType.DMA,
        pltpu.SemaphoreType.DMA,
    ],
)(_sc_deg_body)

_sc_edge = functools.partial(
    pl.kernel,
    out_type=jax.ShapeDtypeStruct((NC, N_PAD, H), jnp.float32),
    mesh=_MESH,
    compiler_params=_SC_PARAMS,
    scratch_types=[
        pltpu.VMEM_SHARED((N_PAD, H), jnp.float32),
        pltpu.VMEM((2, K, H), jnp.float32),
        pltpu.VMEM((2, BLKB, K), jnp.int32),
        pltpu.VMEM((2, BLKB, K), jnp.int32),
        pltpu.VMEM((2, BLKB, K), jnp.float32),
        pltpu.SemaphoreType.DMA,
        pltpu.SemaphoreType.DMA,
        pltpu.SemaphoreType.DMA,
        pltpu.SemaphoreType.DMA,
        pltpu.SemaphoreType.DMA,
    ],
)(_sc_edge_body)


def kernel(x, edge_index, edge_weight, W_in, b_in, W_gcn, b_gcn, W_out, b_out):
    x_p = jnp.pad(x, ((0, N_PAD - N), (0, 0)))
    pad_e = E_PAD - edge_index.shape[1]
    row_p = jnp.concatenate(
        [edge_index[0], jnp.zeros((pad_e,), jnp.int32)]).reshape(E2D_ROWS, K)
    col_p = jnp.concatenate(
        [edge_index[1], jnp.zeros((pad_e,), jnp.int32)]).reshape(E2D_ROWS, K)
    ew_p = jnp.concatenate(
        [edge_weight, jnp.zeros((pad_e,), jnp.float32)]).reshape(E2D_ROWS, K)

    BR = 1024

    h2 = pl.pallas_call(
        _dense_in_body,
        grid=(N_PAD // BR,),
        in_specs=[
            pl.BlockSpec((BR, D), lambda i: (i, 0)),
            pl.BlockSpec((D, H), lambda i: (0, 0)),
            pl.BlockSpec((1, H), lambda i: (0, 0)),
            pl.BlockSpec((H, H), lambda i: (0, 0)),
        ],
        out_specs=pl.BlockSpec((BR, H), lambda i: (i, 0)),
        out_shape=jax.ShapeDtypeStruct((N_PAD, H), jnp.float32),
    )(x_p, W_in, b_in.reshape(1, H), W_gcn)

    degp = _sc_deg(col_p, ew_p)

    dinv, g = pl.pallas_call(
        _norm_body,
        grid=(N_PAD // BR,),
        in_specs=[
            pl.BlockSpec((NC, BR, 1), lambda i: (0, i, 0)),
            pl.BlockSpec((BR, H), lambda i: (i, 0)),
        ],
        out_specs=[
            pl.BlockSpec((BR, 1), lambda i: (i, 0)),
            pl.BlockSpec((BR, H), lambda i: (i, 0)),
        ],
        out_shape=[
            jax.ShapeDtypeStruct((N_PAD, 1), jnp.float32),
            jax.ShapeDtypeStruct((N_PAD, H), jnp.float32),
        ],
    )(degp.reshape(NC, N_PAD, 1), h2)

    acc = _sc_edge(row_p, col_p, ew_p, g)

    out = pl.pallas_call(
        _combine_body,
        grid=(N_PAD // BR,),
        in_specs=[
            pl.BlockSpec((NC, BR, H), lambda i: (0, i, 0)),
            pl.BlockSpec((BR, H), lambda i: (i, 0)),
            pl.BlockSpec((BR, 1), lambda i: (i, 0)),
            pl.BlockSpec((1, H), lambda i: (0, 0)),
            pl.BlockSpec((H, 40), lambda i: (0, 0)),
            pl.BlockSpec((1, 40), lambda i: (0, 0)),
        ],
        out_specs=pl.BlockSpec((BR, 40), lambda i: (i, 0)),
        out_shape=jax.ShapeDtypeStruct((N_PAD, 40), jnp.float32),
    )(acc, g, dinv, b_gcn.reshape(1, H), W_out, b_out.reshape(1, 40))

    return out[:N]
